# baseline (speedup 1.0000x reference)
import functools

import jax
import jax.numpy as jnp
from jax import lax
from jax.experimental import pallas as pl
from jax.experimental.pallas import tpu as pltpu
from jax.experimental.pallas import tpu_sc as plsc

_NUM_CORES = 2
_NUM_SUBCORES = 16
_NW = _NUM_CORES * _NUM_SUBCORES
_LANES = 16


def _sc_build_rel(tt, h, w):
    c = tt.shape[0]
    hw = h * w
    c_per_w = c // _NW
    mesh = plsc.VectorSubcoreMesh(core_axis_name="c", subcore_axis_name="s")

    @functools.partial(
        pl.kernel,
        out_type=jax.ShapeDtypeStruct((c, hw), jnp.float32),
        mesh=mesh,
        scratch_types=[
            pltpu.VMEM((128,), jnp.float32),
            pltpu.VMEM((hw,), jnp.float32),
        ],
    )
    def rel_kernel(tt_hbm, rel_hbm, tcol_v, row_v):
        wid = lax.axis_index("s") * _NUM_CORES + lax.axis_index("c")
        for ci in range(c_per_w):
            ch = wid * c_per_w + ci
            pltpu.sync_copy(tt_hbm.at[ch], tcol_v)

            def body(hh, carry):
                off = (w - 1) - hh
                for k in range(w // _LANES):
                    row_v[pl.ds(hh * w + k * _LANES, _LANES)] = (
                        tcol_v[pl.ds(off + k * _LANES, _LANES)]
                    )
                return carry

            lax.fori_loop(0, h, body, 0)
            pltpu.sync_copy(row_v, rel_hbm.at[ch])

    return rel_kernel(tt)


def _tc_add_body(x_ref, rel_ref, o_ref):
    o_ref[...] = x_ref[...] + rel_ref[...]


def _tc_add(x3, rel):
    b, c, hw = x3.shape
    tile_c = 64
    grid = (c // tile_c, b)
    return pl.pallas_call(
        _tc_add_body,
        grid=grid,
        in_specs=[
            pl.BlockSpec((1, tile_c, hw), lambda ci, bi: (bi, ci, 0)),
            pl.BlockSpec((tile_c, hw), lambda ci, bi: (ci, 0)),
        ],
        out_specs=pl.BlockSpec((1, tile_c, hw), lambda ci, bi: (bi, ci, 0)),
        out_shape=jax.ShapeDtypeStruct((b, c, hw), jnp.float32),
    )(x3, rel)


def kernel(x, rel_emb_x, rel_emb_y):
    b, c, h, w = x.shape
    t = jnp.concatenate([rel_emb_x, rel_emb_y], axis=1)
    tt = jnp.pad(t.T, ((0, 0), (0, 128 - t.shape[0])))
    rel = _sc_build_rel(tt, h, w)
    out = _tc_add(x.reshape(b, c, h * w), rel)
    return out.reshape(b, c, h, w)
